# baseline (speedup 1.0000x reference)
import functools

import jax
import jax.numpy as jnp
from jax import lax
from jax.experimental import pallas as pl
from jax.experimental.pallas import tpu as pltpu
from jax.experimental.pallas import tpu_sc as plsc

N = 10000
E = 320000
R = 2
D = 128
DH = 64
DO = 16
DW = DH + DO
TN = 2 * N
SUB = 128
NSC = 2
NT = 16
ROWS_PER_TILE = 1280
TNP = NT * ROWS_PER_TILE

CHI = 8
SUBS1 = 80
NCHI1 = SUBS1 // CHI
EPT1 = SUBS1 * SUB
EPAD = NSC * NT * EPT1
EROWS = EPAD // SUB
CH2 = 4
SUBS2 = 80
NCH2 = SUBS2 // CH2
EPT2 = SUBS2 * SUB
assert NSC * NT * EPT2 == EPAD


def _dense1_body(x_ref, basis_ref, comp_ref, root_ref, bias_ref, y_ref, self_ref):
    xb = x_ref[...]
    for r in range(R):
        w = comp_ref[r, 0] * basis_ref[0]
        for b in range(1, 10):
            w = w + comp_ref[r, b] * basis_ref[b]
        t = jnp.dot(xb, w, preferred_element_type=jnp.float32)
        y_ref[r] = t.astype(jnp.bfloat16)
    self_ref[...] = (
        jnp.dot(xb, root_ref[...], preferred_element_type=jnp.float32)
        + bias_ref[...]
    )


def _dense1(x, basis1, comp1, root1, bias1, blk):
    grid = N // blk
    return pl.pallas_call(
        _dense1_body,
        grid=(grid,),
        in_specs=[
            pl.BlockSpec((blk, D), lambda i: (i, 0)),
            pl.BlockSpec((10, D, D), lambda i: (0, 0, 0)),
            pl.BlockSpec(memory_space=pltpu.SMEM),
            pl.BlockSpec((D, D), lambda i: (0, 0)),
            pl.BlockSpec((1, D), lambda i: (0, 0)),
        ],
        out_specs=[
            pl.BlockSpec((R, blk, D), lambda i: (0, i, 0)),
            pl.BlockSpec((blk, D), lambda i: (i, 0)),
        ],
        out_shape=[
            jax.ShapeDtypeStruct((R, N, D), jnp.bfloat16),
            jax.ShapeDtypeStruct((N, D), jnp.float32),
        ],
    )(x, basis1, comp1, root1, bias1.reshape(1, D))


def _sc_count_body(sidx2, z16, o16, cnt_out, acc, sidx_t, ones_v, sa, sb):
    sem = (sa, sb)
    cid = lax.axis_index("c")
    sid = lax.axis_index("s")
    row0 = sid * ROWS_PER_TILE
    brow = (cid * NT + sid) * SUBS2
    pltpu.sync_copy(z16, acc.at[pl.ds(row0, ROWS_PER_TILE)])
    pltpu.sync_copy(o16, ones_v)
    pltpu.sync_copy(sidx2.at[pl.ds(brow, SUBS2)], sidx_t)
    plsc.subcore_barrier()

    def wait8(s):
        for _ in range(8):
            pltpu.make_async_copy(z16.at[pl.ds(0, SUB)], ones_v, s).wait()

    for t in range(SUBS2 // 8):
        for j in range(8):
            pltpu.async_copy(ones_v, acc.at[sidx_t.at[t * 8 + j]],
                             sem[t % 2], add=True)
        if t >= 1:
            wait8(sem[(t - 1) % 2])
    wait8(sem[(SUBS2 // 8 - 1) % 2])
    plsc.subcore_barrier()

    pltpu.sync_copy(acc.at[pl.ds(row0, ROWS_PER_TILE)],
                    cnt_out.at[pl.ds(cid * TNP + row0, ROWS_PER_TILE)])


def _sc_count(sidx2, z16, o16):
    mesh = plsc.VectorSubcoreMesh(core_axis_name="c", subcore_axis_name="s")
    f = pl.kernel(
        _sc_count_body,
        out_type=jax.ShapeDtypeStruct((NSC * TNP, DO), jnp.float32),
        mesh=mesh,
        compiler_params=pltpu.CompilerParams(use_tc_tiling_on_sc=False),
        scratch_types=[
            pltpu.VMEM_SHARED((TNP, DO), jnp.float32),
            pltpu.VMEM((SUBS2, SUB), jnp.int32),
            pltpu.VMEM((SUB, DO), jnp.float32),
            pltpu.SemaphoreType.DMA,
            pltpu.SemaphoreType.DMA,
        ],
    )
    return f(sidx2, z16, o16)


def _sc_layer1_body(table, g2, sidx2, z64, s1_out,
                    acc, gibuf, sibuf, rows,
                    gA, gB, gC, gD, sA, sB, sC, sD, isem):
    gsem = (gA, gB, gC, gD)
    ssem = (sA, sB, sC, sD)
    cid = lax.axis_index("c")
    sid = lax.axis_index("s")
    row0 = sid * ROWS_PER_TILE
    brow = (cid * NT + sid) * SUBS1
    pltpu.sync_copy(z64, acc.at[pl.ds(row0, ROWS_PER_TILE)])
    plsc.subcore_barrier()

    def stage(c, p):
        pltpu.async_copy(g2.at[pl.ds(brow + c * CHI, CHI)], gibuf.at[p], isem)
        pltpu.async_copy(sidx2.at[pl.ds(brow + c * CHI, CHI)], sibuf.at[p],
                         isem)

    def wait_i():
        pltpu.make_async_copy(g2.at[pl.ds(0, CHI)], gibuf.at[0], isem).wait()
        pltpu.make_async_copy(g2.at[pl.ds(0, CHI)], sibuf.at[0], isem).wait()

    def fire_g(p, j8, b):
        pltpu.async_copy(table.at[gibuf.at[p, j8]], rows.at[b], gsem[b])

    def wait_g(b):
        pltpu.make_async_copy(table.at[pl.ds(0, SUB)], rows.at[b],
                              gsem[b]).wait()

    def fire_s(p, j8, b):
        pltpu.async_copy(rows.at[b], acc.at[sibuf.at[p, j8]], ssem[b],
                         add=True)

    def wait_s(b):
        pltpu.make_async_copy(table.at[pl.ds(0, SUB)], rows.at[b],
                              ssem[b]).wait()


    pltpu.sync_copy(g2.at[pl.ds(brow, CHI)], gibuf.at[0])
    pltpu.sync_copy(sidx2.at[pl.ds(brow, CHI)], sibuf.at[0])
    stage(1, 1)
    for b in range(4):
        fire_g(0, b, b)
    for b in range(3):
        wait_g(b)
        fire_s(0, b, b)
    for g in range(4, 2 * CHI):
        b = g % 4
        j8 = g % CHI
        p = g // CHI
        if j8 == 0:
            wait_i()
        wait_s(b)
        fire_g(p, j8, b)
        wait_g((g - 1) % 4)
        fire_s((g - 1) // CHI, (g - 1) % CHI, (g - 1) % 4)
        if g == 11:
            stage(2, 0)

    def body(kk, carry):
        c0 = 2 * kk + 2
        for j in range(2 * CHI):
            b = j % 4
            j8 = j % CHI
            p = 0 if j < CHI else 1
            if j8 == 0:
                wait_i()
            wait_s(b)
            fire_g(p, j8, b)
            wait_g((j - 1) % 4)
            pj = p if j8 != 0 else 1 - p
            fire_s(pj, (j8 - 1) % CHI, (j - 1) % 4)
            if j == 3:
                stage(c0 + 1, 1)
            elif j == 11:
                stage(c0 + 2, 0)
        return carry

    lax.fori_loop(0, (NCHI1 - 2) // 2, body, 0)

    wait_g(3)
    fire_s(1, CHI - 1, 3)
    for b in range(4):
        wait_s(b)
    wait_i()
    plsc.subcore_barrier()

    pltpu.sync_copy(acc.at[pl.ds(row0, ROWS_PER_TILE)],
                    s1_out.at[pl.ds(cid * TNP + row0, ROWS_PER_TILE)])


def _sc_layer1(table, g2, sidx2, z64):
    mesh = plsc.VectorSubcoreMesh(core_axis_name="c", subcore_axis_name="s")
    f = pl.kernel(
        _sc_layer1_body,
        out_type=jax.ShapeDtypeStruct((NSC * TNP, D), jnp.bfloat16),
        mesh=mesh,
        compiler_params=pltpu.CompilerParams(use_tc_tiling_on_sc=False),
        scratch_types=[
            pltpu.VMEM_SHARED((TNP, D), jnp.bfloat16),
            pltpu.VMEM((2, CHI, SUB), jnp.int32),
            pltpu.VMEM((2, CHI, SUB), jnp.int32),
            pltpu.VMEM((4, SUB, D), jnp.bfloat16),
            pltpu.SemaphoreType.DMA,
            pltpu.SemaphoreType.DMA,
            pltpu.SemaphoreType.DMA,
            pltpu.SemaphoreType.DMA,
            pltpu.SemaphoreType.DMA,
            pltpu.SemaphoreType.DMA,
            pltpu.SemaphoreType.DMA,
            pltpu.SemaphoreType.DMA,
            pltpu.SemaphoreType.DMA,
        ],
    )
    return f(table, g2, sidx2, z64)


def _dense2_body(self1_ref, s00, s01, s10, s11, c00, c01, c10, c11,
                 basis_ref, comp_ref, root_ref, bias_ref,
                 z_ref, self2_ref, inv_ref):
    cnt0 = c00[0][:, 0:1] + c10[0][:, 0:1]
    cnt1 = c01[0][:, 0:1] + c11[0][:, 0:1]
    inv0 = 1.0 / jnp.maximum(cnt0, 1.0)
    inv1 = 1.0 / jnp.maximum(cnt1, 1.0)
    agg0 = (s00[0].astype(jnp.float32) + s10[0].astype(jnp.float32)) * inv0
    agg1 = (s01[0].astype(jnp.float32) + s11[0].astype(jnp.float32)) * inv1
    h = jnp.maximum(self1_ref[...] + agg0 + agg1, 0.0)
    for r in range(R):
        w = comp_ref[r, 0] * basis_ref[0]
        for b in range(1, 10):
            w = w + comp_ref[r, b] * basis_ref[b]
        z_ref[r] = jnp.dot(h, w, preferred_element_type=jnp.float32).astype(
            jnp.bfloat16)
    self2_ref[...] = (
        jnp.dot(h, root_ref[...], preferred_element_type=jnp.float32)
        + bias_ref[...]
    )
    n = inv0.shape[0]
    inv_ref[...] = jnp.concatenate(
        [jnp.broadcast_to(inv0, (n, DO)), jnp.broadcast_to(inv1, (n, DO))],
        axis=1)


def _dense2(self1, s1, cnt, basis2, comp2, root2, bias2, blk):
    grid = N // blk
    nb = N // blk
    s1r = s1.reshape(NSC, TNP, D)
    cr = cnt.reshape(NSC, TNP, DO)
    sspec = lambda k: pl.BlockSpec(
        (1, blk, D), lambda i, k=k: (k // 2, (k % 2) * nb + i, 0))
    cspec = lambda k: pl.BlockSpec(
        (1, blk, DO), lambda i, k=k: (k // 2, (k % 2) * nb + i, 0))
    return pl.pallas_call(
        _dense2_body,
        grid=(grid,),
        in_specs=[
            pl.BlockSpec((blk, D), lambda i: (i, 0)),
            sspec(0), sspec(1), sspec(2), sspec(3),
            cspec(0), cspec(1), cspec(2), cspec(3),
            pl.BlockSpec((10, D, DO), lambda i: (0, 0, 0)),
            pl.BlockSpec(memory_space=pltpu.SMEM),
            pl.BlockSpec((D, DO), lambda i: (0, 0)),
            pl.BlockSpec((1, DO), lambda i: (0, 0)),
        ],
        out_specs=[
            pl.BlockSpec((R, blk, DO), lambda i: (0, i, 0)),
            pl.BlockSpec((blk, DO), lambda i: (i, 0)),
            pl.BlockSpec((blk, 2 * DO), lambda i: (i, 0)),
        ],
        out_shape=[
            jax.ShapeDtypeStruct((R, N, DO), jnp.bfloat16),
            jax.ShapeDtypeStruct((N, DO), jnp.float32),
            jax.ShapeDtypeStruct((N, 2 * DO), jnp.float32),
        ],
    )(self1, s1r, s1r, s1r, s1r, cr, cr, cr, cr,
      basis2, comp2, root2, bias2.reshape(1, DO))


def _sc_layer2_body(table, gidx2, sidx2, z16, s2_out,
                    acc, gidx_t, sidx_t, rows, gs0, gs1, ss0, ss1):
    gsem = (gs0, gs1)
    ssem = (ss0, ss1)
    cid = lax.axis_index("c")
    sid = lax.axis_index("s")
    row0 = sid * ROWS_PER_TILE
    brow = (cid * NT + sid) * SUBS2
    pltpu.sync_copy(z16, acc.at[pl.ds(row0, ROWS_PER_TILE)])
    pltpu.sync_copy(gidx2.at[pl.ds(brow, SUBS2)], gidx_t)
    pltpu.sync_copy(sidx2.at[pl.ds(brow, SUBS2)], sidx_t)
    plsc.subcore_barrier()

    def fire_g(ch, r):
        for b in range(CH2):
            pltpu.async_copy(table.at[gidx_t.at[ch * CH2 + b]],
                             rows.at[r, b], gsem[r])

    def wait_g(r):
        for b in range(CH2):
            pltpu.make_async_copy(table.at[pl.ds(0, SUB)],
                                  rows.at[r, b], gsem[r]).wait()

    def fire_s(ch, r):
        for b in range(CH2):
            pltpu.async_copy(rows.at[r, b], acc.at[sidx_t.at[ch * CH2 + b]],
                             ssem[r], add=True)

    def wait_s(r):
        for b in range(CH2):
            pltpu.make_async_copy(table.at[pl.ds(0, SUB)],
                                  rows.at[r, b], ssem[r]).wait()

    fire_g(0, 0)
    fire_g(1, 1)
    wait_g(0)
    fire_s(0, 0)

    def body(kk, carry):
        ch0 = 2 * kk + 2
        wait_s(0)
        fire_g(ch0, 0)
        wait_g(1)
        fire_s(ch0 - 1, 1)
        wait_s(1)
        fire_g(ch0 + 1, 1)
        wait_g(0)
        fire_s(ch0, 0)
        return carry

    lax.fori_loop(0, (NCH2 - 2) // 2, body, 0)
    wait_g(1)
    fire_s(NCH2 - 1, 1)
    wait_s(0)
    wait_s(1)
    plsc.subcore_barrier()

    pltpu.sync_copy(acc.at[pl.ds(row0, ROWS_PER_TILE)],
                    s2_out.at[pl.ds(cid * TNP + row0, ROWS_PER_TILE)])


def _sc_layer2(table, gidx2, sidx2, z16):
    mesh = plsc.VectorSubcoreMesh(core_axis_name="c", subcore_axis_name="s")
    f = pl.kernel(
        _sc_layer2_body,
        out_type=jax.ShapeDtypeStruct((NSC * TNP, DO), jnp.bfloat16),
        mesh=mesh,
        compiler_params=pltpu.CompilerParams(use_tc_tiling_on_sc=False),
        scratch_types=[
            pltpu.VMEM_SHARED((TNP, DO), jnp.bfloat16),
            pltpu.VMEM((SUBS2, SUB), jnp.int32),
            pltpu.VMEM((SUBS2, SUB), jnp.int32),
            pltpu.VMEM((2, CH2, SUB, DO), jnp.bfloat16),
            pltpu.SemaphoreType.DMA,
            pltpu.SemaphoreType.DMA,
            pltpu.SemaphoreType.DMA,
            pltpu.SemaphoreType.DMA,
        ],
    )
    return f(table, gidx2, sidx2, z16)


def _final_body(self2_ref, p00, p01, p10, p11, inv_ref, out_ref):
    inv0 = inv_ref[:, 0:1]
    inv1 = inv_ref[:, DO:DO + 1]
    agg0 = (p00[0].astype(jnp.float32) + p10[0].astype(jnp.float32)) * inv0
    agg1 = (p01[0].astype(jnp.float32) + p11[0].astype(jnp.float32)) * inv1
    o = self2_ref[...] + agg0 + agg1
    m = jnp.max(o, axis=1, keepdims=True)
    lse = jnp.log(jnp.sum(jnp.exp(o - m), axis=1, keepdims=True))
    out_ref[...] = o - m - lse


def _final(self2, s2p, inv, blk):
    grid = N // blk
    nb = N // blk
    s2r = s2p.reshape(NSC, TNP, DO)
    pspec = lambda k: pl.BlockSpec(
        (1, blk, DO), lambda i, k=k: (k // 2, (k % 2) * nb + i, 0))
    return pl.pallas_call(
        _final_body,
        grid=(grid,),
        in_specs=[
            pl.BlockSpec((blk, DO), lambda i: (i, 0)),
            pspec(0), pspec(1), pspec(2), pspec(3),
            pl.BlockSpec((blk, 2 * DO), lambda i: (i, 0)),
        ],
        out_specs=pl.BlockSpec((blk, DO), lambda i: (i, 0)),
        out_shape=jax.ShapeDtypeStruct((N, DO), jnp.float32),
    )(self2, s2r, s2r, s2r, s2r, inv)


def kernel(x, edge_index, edge_type, basis1, comp1, root1, bias1,
           basis2, comp2, root2, bias2):
    src = edge_index[0]
    dst = edge_index[1]
    et = edge_type
    gidx = et * N + src
    sidx = et * N + dst
    npad = EPAD - E
    xpad = CHI * SUB
    gidx_pad = jnp.concatenate([gidx, jnp.zeros((npad,), jnp.int32)])
    sidx_pad = jnp.concatenate([sidx, jnp.full((npad,), TN, jnp.int32)])
    gidx_2d = jnp.concatenate(
        [gidx_pad, jnp.zeros((xpad,), jnp.int32)]
    ).reshape(EROWS + CHI, SUB)
    sidx_2d = jnp.concatenate(
        [sidx_pad, jnp.full((xpad,), TN, jnp.int32)]
    ).reshape(EROWS + CHI, SUB)

    z128 = jnp.zeros((ROWS_PER_TILE, D), jnp.bfloat16)
    z16 = jnp.zeros((ROWS_PER_TILE, DO), jnp.float32)
    z16b = jnp.zeros((ROWS_PER_TILE, DO), jnp.bfloat16)
    o16 = jnp.ones((SUB, DO), jnp.float32)

    cnt = _sc_count(sidx_2d, z16, o16)
    y1, self1 = _dense1(x, basis1, comp1, root1, bias1, blk=2000)
    table1 = y1.reshape(TN, D)
    s1 = _sc_layer1(table1, gidx_2d, sidx_2d, z128)
    z, self2, inv = _dense2(self1, s1, cnt, basis2, comp2, root2, bias2,
                            blk=2000)
    table2 = z.reshape(TN, DO)
    s2p = _sc_layer2(table2, gidx_2d, sidx_2d, z16b)
    return _final(self2, s2p, inv, blk=2000)


# speedup vs baseline: 1.3037x; 1.3037x over previous
import functools

import jax
import jax.numpy as jnp
from jax import lax
from jax.experimental import pallas as pl
from jax.experimental.pallas import tpu as pltpu
from jax.experimental.pallas import tpu_sc as plsc

N = 10000
E = 320000
R = 2
D = 128
DH = 64
DO = 16
DW = DH + DO
TN = 2 * N
SUB = 128
NSC = 2
NT = 16
ROWS_PER_TILE = 1280
TNP = NT * ROWS_PER_TILE

CHI = 8
SUBS1 = 160
NCHI1 = SUBS1 // CHI
EPT1 = SUBS1 * SUB
EPAD = NT * EPT1
EROWS = EPAD // SUB
CH2 = 4
SUBS2 = 80
NCH2 = SUBS2 // CH2
EPT2 = SUBS2 * SUB
assert NSC * NT * EPT2 == EPAD


def _dense1_body(x_ref, basis_ref, comp_ref, root_ref, bias_ref, y_ref, self_ref):
    xb = x_ref[...]
    for r in range(R):
        w = comp_ref[r, 0] * basis_ref[0]
        for b in range(1, 10):
            w = w + comp_ref[r, b] * basis_ref[b]
        t = jnp.dot(xb, w, preferred_element_type=jnp.float32)
        tb = t.astype(jnp.bfloat16)
        y_ref[0, r] = tb[:, :DH]
        y_ref[1, r] = tb[:, DH:]
    self_ref[...] = (
        jnp.dot(xb, root_ref[...], preferred_element_type=jnp.float32)
        + bias_ref[...]
    )


def _dense1(x, basis1, comp1, root1, bias1, blk):
    grid = N // blk
    return pl.pallas_call(
        _dense1_body,
        grid=(grid,),
        in_specs=[
            pl.BlockSpec((blk, D), lambda i: (i, 0)),
            pl.BlockSpec((10, D, D), lambda i: (0, 0, 0)),
            pl.BlockSpec(memory_space=pltpu.SMEM),
            pl.BlockSpec((D, D), lambda i: (0, 0)),
            pl.BlockSpec((1, D), lambda i: (0, 0)),
        ],
        out_specs=[
            pl.BlockSpec((NSC, R, blk, DH), lambda i: (0, 0, i, 0)),
            pl.BlockSpec((blk, D), lambda i: (i, 0)),
        ],
        out_shape=[
            jax.ShapeDtypeStruct((NSC, R, N, DH), jnp.bfloat16),
            jax.ShapeDtypeStruct((N, D), jnp.float32),
        ],
    )(x, basis1, comp1, root1, bias1.reshape(1, D))


def _sc_count_body(sidx2, z16, o16, cnt_out, acc, sidx_t, ones_v, sa, sb):
    sem = (sa, sb)
    cid = lax.axis_index("c")
    sid = lax.axis_index("s")
    row0 = sid * ROWS_PER_TILE
    brow = (cid * NT + sid) * SUBS2
    pltpu.sync_copy(z16, acc.at[pl.ds(row0, ROWS_PER_TILE)])
    pltpu.sync_copy(o16, ones_v)
    pltpu.sync_copy(sidx2.at[pl.ds(brow, SUBS2)], sidx_t)
    plsc.subcore_barrier()

    def wait8(s):
        for _ in range(8):
            pltpu.make_async_copy(z16.at[pl.ds(0, SUB)], ones_v, s).wait()

    for t in range(SUBS2 // 8):
        for j in range(8):
            pltpu.async_copy(ones_v, acc.at[sidx_t.at[t * 8 + j]],
                             sem[t % 2], add=True)
        if t >= 1:
            wait8(sem[(t - 1) % 2])
    wait8(sem[(SUBS2 // 8 - 1) % 2])
    plsc.subcore_barrier()

    pltpu.sync_copy(acc.at[pl.ds(row0, ROWS_PER_TILE)],
                    cnt_out.at[pl.ds(cid * TNP + row0, ROWS_PER_TILE)])


def _sc_count(sidx2, z16, o16):
    mesh = plsc.VectorSubcoreMesh(core_axis_name="c", subcore_axis_name="s")
    f = pl.kernel(
        _sc_count_body,
        out_type=jax.ShapeDtypeStruct((NSC * TNP, DO), jnp.float32),
        mesh=mesh,
        compiler_params=pltpu.CompilerParams(use_tc_tiling_on_sc=False),
        scratch_types=[
            pltpu.VMEM_SHARED((TNP, DO), jnp.float32),
            pltpu.VMEM((SUBS2, SUB), jnp.int32),
            pltpu.VMEM((SUB, DO), jnp.float32),
            pltpu.SemaphoreType.DMA,
            pltpu.SemaphoreType.DMA,
        ],
    )
    return f(sidx2, z16, o16)


def _sc_layer1_body(table, g2, sidx2, z64, s1_out,
                    acc, gibuf, sibuf, rows,
                    gA, gB, gC, gD, sA, sB, sC, sD, isem):
    gsem = (gA, gB, gC, gD)
    ssem = (sA, sB, sC, sD)
    cid = lax.axis_index("c")
    sid = lax.axis_index("s")
    row0 = sid * ROWS_PER_TILE
    gbase = cid * EROWS + sid * SUBS1
    sbase = sid * SUBS1
    pltpu.sync_copy(z64, acc.at[pl.ds(row0, ROWS_PER_TILE)])
    plsc.subcore_barrier()

    def stage(c, p):
        pltpu.async_copy(g2.at[pl.ds(gbase + c * CHI, CHI)], gibuf.at[p], isem)
        pltpu.async_copy(sidx2.at[pl.ds(sbase + c * CHI, CHI)], sibuf.at[p],
                         isem)

    def wait_i():
        pltpu.make_async_copy(g2.at[pl.ds(0, CHI)], gibuf.at[0], isem).wait()
        pltpu.make_async_copy(g2.at[pl.ds(0, CHI)], sibuf.at[0], isem).wait()

    def fire_g(p, j8, b):
        pltpu.async_copy(table.at[gibuf.at[p, j8]], rows.at[b], gsem[b])

    def wait_g(b):
        pltpu.make_async_copy(table.at[pl.ds(0, SUB)], rows.at[b],
                              gsem[b]).wait()

    def fire_s(p, j8, b):
        pltpu.async_copy(rows.at[b], acc.at[sibuf.at[p, j8]], ssem[b],
                         add=True)

    def wait_s(b):
        pltpu.make_async_copy(table.at[pl.ds(0, SUB)], rows.at[b],
                              ssem[b]).wait()


    pltpu.sync_copy(g2.at[pl.ds(gbase, CHI)], gibuf.at[0])
    pltpu.sync_copy(sidx2.at[pl.ds(sbase, CHI)], sibuf.at[0])
    stage(1, 1)
    for b in range(4):
        fire_g(0, b, b)
    for b in range(3):
        wait_g(b)
        fire_s(0, b, b)
    for g in range(4, 2 * CHI):
        b = g % 4
        j8 = g % CHI
        p = g // CHI
        if j8 == 0:
            wait_i()
        wait_s(b)
        fire_g(p, j8, b)
        wait_g((g - 1) % 4)
        fire_s((g - 1) // CHI, (g - 1) % CHI, (g - 1) % 4)
        if g == 11:
            stage(2, 0)

    def body(kk, carry):
        c0 = 2 * kk + 2
        for j in range(2 * CHI):
            b = j % 4
            j8 = j % CHI
            p = 0 if j < CHI else 1
            if j8 == 0:
                wait_i()
            wait_s(b)
            fire_g(p, j8, b)
            wait_g((j - 1) % 4)
            pj = p if j8 != 0 else 1 - p
            fire_s(pj, (j8 - 1) % CHI, (j - 1) % 4)
            if j == 3:
                stage(c0 + 1, 1)
            elif j == 11:
                stage(c0 + 2, 0)
        return carry

    lax.fori_loop(0, (NCHI1 - 2) // 2, body, 0)

    wait_g(3)
    fire_s(1, CHI - 1, 3)
    for b in range(4):
        wait_s(b)
    wait_i()
    plsc.subcore_barrier()

    pltpu.sync_copy(acc.at[pl.ds(row0, ROWS_PER_TILE)],
                    s1_out.at[pl.ds(cid * TNP + row0, ROWS_PER_TILE)])


def _sc_layer1(table, g2, sidx2, z64):
    mesh = plsc.VectorSubcoreMesh(core_axis_name="c", subcore_axis_name="s")
    f = pl.kernel(
        _sc_layer1_body,
        out_type=jax.ShapeDtypeStruct((NSC * TNP, DH), jnp.bfloat16),
        mesh=mesh,
        compiler_params=pltpu.CompilerParams(use_tc_tiling_on_sc=False),
        scratch_types=[
            pltpu.VMEM_SHARED((TNP, DH), jnp.bfloat16),
            pltpu.VMEM((2, CHI, SUB), jnp.int32),
            pltpu.VMEM((2, CHI, SUB), jnp.int32),
            pltpu.VMEM((4, SUB, DH), jnp.bfloat16),
            pltpu.SemaphoreType.DMA,
            pltpu.SemaphoreType.DMA,
            pltpu.SemaphoreType.DMA,
            pltpu.SemaphoreType.DMA,
            pltpu.SemaphoreType.DMA,
            pltpu.SemaphoreType.DMA,
            pltpu.SemaphoreType.DMA,
            pltpu.SemaphoreType.DMA,
            pltpu.SemaphoreType.DMA,
        ],
    )
    return f(table, g2, sidx2, z64)


def _dense2_body(self1_ref, s00, s01, s10, s11, c00, c01, c10, c11,
                 basis_ref, comp_ref, root_ref, bias_ref,
                 z_ref, self2_ref, inv_ref):
    cnt0 = c00[0][:, 0:1] + c10[0][:, 0:1]
    cnt1 = c01[0][:, 0:1] + c11[0][:, 0:1]
    inv0 = 1.0 / jnp.maximum(cnt0, 1.0)
    inv1 = 1.0 / jnp.maximum(cnt1, 1.0)
    agg0 = jnp.concatenate([s00[0], s10[0]], axis=1).astype(jnp.float32) * inv0
    agg1 = jnp.concatenate([s01[0], s11[0]], axis=1).astype(jnp.float32) * inv1
    h = jnp.maximum(self1_ref[...] + agg0 + agg1, 0.0)
    for r in range(R):
        w = comp_ref[r, 0] * basis_ref[0]
        for b in range(1, 10):
            w = w + comp_ref[r, b] * basis_ref[b]
        z_ref[r] = jnp.dot(h, w, preferred_element_type=jnp.float32).astype(
            jnp.bfloat16)
    self2_ref[...] = (
        jnp.dot(h, root_ref[...], preferred_element_type=jnp.float32)
        + bias_ref[...]
    )
    n = inv0.shape[0]
    inv_ref[...] = jnp.concatenate(
        [jnp.broadcast_to(inv0, (n, DO)), jnp.broadcast_to(inv1, (n, DO))],
        axis=1)


def _dense2(self1, s1, cnt, basis2, comp2, root2, bias2, blk):
    grid = N // blk
    nb = N // blk
    s1r = s1.reshape(NSC, TNP, DH)
    cr = cnt.reshape(NSC, TNP, DO)
    sspec = lambda k: pl.BlockSpec(
        (1, blk, DH), lambda i, k=k: (k // 2, (k % 2) * nb + i, 0))
    cspec = lambda k: pl.BlockSpec(
        (1, blk, DO), lambda i, k=k: (k // 2, (k % 2) * nb + i, 0))
    return pl.pallas_call(
        _dense2_body,
        grid=(grid,),
        in_specs=[
            pl.BlockSpec((blk, D), lambda i: (i, 0)),
            sspec(0), sspec(1), sspec(2), sspec(3),
            cspec(0), cspec(1), cspec(2), cspec(3),
            pl.BlockSpec((10, D, DO), lambda i: (0, 0, 0)),
            pl.BlockSpec(memory_space=pltpu.SMEM),
            pl.BlockSpec((D, DO), lambda i: (0, 0)),
            pl.BlockSpec((1, DO), lambda i: (0, 0)),
        ],
        out_specs=[
            pl.BlockSpec((R, blk, DO), lambda i: (0, i, 0)),
            pl.BlockSpec((blk, DO), lambda i: (i, 0)),
            pl.BlockSpec((blk, 2 * DO), lambda i: (i, 0)),
        ],
        out_shape=[
            jax.ShapeDtypeStruct((R, N, DO), jnp.bfloat16),
            jax.ShapeDtypeStruct((N, DO), jnp.float32),
            jax.ShapeDtypeStruct((N, 2 * DO), jnp.float32),
        ],
    )(self1, s1r, s1r, s1r, s1r, cr, cr, cr, cr,
      basis2, comp2, root2, bias2.reshape(1, DO))


def _sc_layer2_body(table, gidx2, sidx2, z16, s2_out,
                    acc, gidx_t, sidx_t, rows, gs0, gs1, ss0, ss1):
    gsem = (gs0, gs1)
    ssem = (ss0, ss1)
    cid = lax.axis_index("c")
    sid = lax.axis_index("s")
    row0 = sid * ROWS_PER_TILE
    brow = (cid * NT + sid) * SUBS2
    pltpu.sync_copy(z16, acc.at[pl.ds(row0, ROWS_PER_TILE)])
    pltpu.sync_copy(gidx2.at[pl.ds(brow, SUBS2)], gidx_t)
    pltpu.sync_copy(sidx2.at[pl.ds(brow, SUBS2)], sidx_t)
    plsc.subcore_barrier()

    def fire_g(ch, r):
        for b in range(CH2):
            pltpu.async_copy(table.at[gidx_t.at[ch * CH2 + b]],
                             rows.at[r, b], gsem[r])

    def wait_g(r):
        for b in range(CH2):
            pltpu.make_async_copy(table.at[pl.ds(0, SUB)],
                                  rows.at[r, b], gsem[r]).wait()

    def fire_s(ch, r):
        for b in range(CH2):
            pltpu.async_copy(rows.at[r, b], acc.at[sidx_t.at[ch * CH2 + b]],
                             ssem[r], add=True)

    def wait_s(r):
        for b in range(CH2):
            pltpu.make_async_copy(table.at[pl.ds(0, SUB)],
                                  rows.at[r, b], ssem[r]).wait()

    fire_g(0, 0)
    fire_g(1, 1)
    wait_g(0)
    fire_s(0, 0)

    def body(kk, carry):
        ch0 = 2 * kk + 2
        wait_s(0)
        fire_g(ch0, 0)
        wait_g(1)
        fire_s(ch0 - 1, 1)
        wait_s(1)
        fire_g(ch0 + 1, 1)
        wait_g(0)
        fire_s(ch0, 0)
        return carry

    lax.fori_loop(0, (NCH2 - 2) // 2, body, 0)
    wait_g(1)
    fire_s(NCH2 - 1, 1)
    wait_s(0)
    wait_s(1)
    plsc.subcore_barrier()

    pltpu.sync_copy(acc.at[pl.ds(row0, ROWS_PER_TILE)],
                    s2_out.at[pl.ds(cid * TNP + row0, ROWS_PER_TILE)])


def _sc_layer2(table, gidx2, sidx2, z16):
    mesh = plsc.VectorSubcoreMesh(core_axis_name="c", subcore_axis_name="s")
    f = pl.kernel(
        _sc_layer2_body,
        out_type=jax.ShapeDtypeStruct((NSC * TNP, DO), jnp.bfloat16),
        mesh=mesh,
        compiler_params=pltpu.CompilerParams(use_tc_tiling_on_sc=False),
        scratch_types=[
            pltpu.VMEM_SHARED((TNP, DO), jnp.bfloat16),
            pltpu.VMEM((SUBS2, SUB), jnp.int32),
            pltpu.VMEM((SUBS2, SUB), jnp.int32),
            pltpu.VMEM((2, CH2, SUB, DO), jnp.bfloat16),
            pltpu.SemaphoreType.DMA,
            pltpu.SemaphoreType.DMA,
            pltpu.SemaphoreType.DMA,
            pltpu.SemaphoreType.DMA,
        ],
    )
    return f(table, gidx2, sidx2, z16)


def _final_body(self2_ref, p00, p01, p10, p11, inv_ref, out_ref):
    inv0 = inv_ref[:, 0:1]
    inv1 = inv_ref[:, DO:DO + 1]
    agg0 = (p00[0].astype(jnp.float32) + p10[0].astype(jnp.float32)) * inv0
    agg1 = (p01[0].astype(jnp.float32) + p11[0].astype(jnp.float32)) * inv1
    o = self2_ref[...] + agg0 + agg1
    m = jnp.max(o, axis=1, keepdims=True)
    lse = jnp.log(jnp.sum(jnp.exp(o - m), axis=1, keepdims=True))
    out_ref[...] = o - m - lse


def _final(self2, s2p, inv, blk):
    grid = N // blk
    nb = N // blk
    s2r = s2p.reshape(NSC, TNP, DO)
    pspec = lambda k: pl.BlockSpec(
        (1, blk, DO), lambda i, k=k: (k // 2, (k % 2) * nb + i, 0))
    return pl.pallas_call(
        _final_body,
        grid=(grid,),
        in_specs=[
            pl.BlockSpec((blk, DO), lambda i: (i, 0)),
            pspec(0), pspec(1), pspec(2), pspec(3),
            pl.BlockSpec((blk, 2 * DO), lambda i: (i, 0)),
        ],
        out_specs=pl.BlockSpec((blk, DO), lambda i: (i, 0)),
        out_shape=jax.ShapeDtypeStruct((N, DO), jnp.float32),
    )(self2, s2r, s2r, s2r, s2r, inv)


def kernel(x, edge_index, edge_type, basis1, comp1, root1, bias1,
           basis2, comp2, root2, bias2):
    src = edge_index[0]
    dst = edge_index[1]
    et = edge_type
    gidx = et * N + src
    sidx = et * N + dst
    npad = EPAD - E
    xpad = CHI * SUB
    gidx_pad = jnp.concatenate([gidx, jnp.zeros((npad,), jnp.int32)])
    sidx_pad = jnp.concatenate([sidx, jnp.full((npad,), TN, jnp.int32)])
    g2 = jnp.concatenate(
        [gidx_pad, gidx_pad + TN, jnp.zeros((xpad,), jnp.int32)]
    ).reshape(2 * EROWS + CHI, SUB)
    gidx_2d = gidx_pad.reshape(EROWS, SUB)
    sidx_2d = jnp.concatenate(
        [sidx_pad, jnp.full((xpad,), TN, jnp.int32)]
    ).reshape(EROWS + CHI, SUB)

    z64 = jnp.zeros((ROWS_PER_TILE, DH), jnp.bfloat16)
    z16 = jnp.zeros((ROWS_PER_TILE, DO), jnp.float32)
    z16b = jnp.zeros((ROWS_PER_TILE, DO), jnp.bfloat16)
    o16 = jnp.ones((SUB, DO), jnp.float32)

    cnt = _sc_count(sidx_2d, z16, o16)
    y1, self1 = _dense1(x, basis1, comp1, root1, bias1, blk=2000)
    table1 = y1.reshape(NSC * TN, DH)
    s1 = _sc_layer1(table1, g2, sidx_2d, z64)
    z, self2, inv = _dense2(self1, s1, cnt, basis2, comp2, root2, bias2,
                            blk=2000)
    table2 = z.reshape(TN, DO)
    s2p = _sc_layer2(table2, gidx_2d, sidx_2d, z16b)
    return _final(self2, s2p, inv, blk=2000)
